# baseline (speedup 1.0000x reference)
import functools
import math

import jax
import jax.numpy as jnp
from jax import lax
from jax.experimental import pallas as pl
from jax.experimental.pallas import tpu as pltpu
from jax.experimental.pallas import tpu_sc as plsc

D_MODEL = 64
SCALE = math.sqrt(D_MODEL)


@functools.lru_cache(maxsize=None)
def _make_gather(V, D, B_total):
    info = plsc.get_sparse_core_info()
    NC, NS, L = info.num_cores, info.num_subcores, info.num_lanes
    NW = NC * NS
    assert B_total % NW == 0
    n_per_w = B_total // NW
    CHUNK = 512
    assert n_per_w % CHUNK == 0
    n_chunks = n_per_w // CHUNK
    mesh = plsc.VectorSubcoreMesh(core_axis_name="c", subcore_axis_name="s")

    @functools.partial(
        pl.kernel,
        out_type=jax.ShapeDtypeStruct((B_total, D), jnp.float32),
        mesh=mesh,
        scratch_types=[
            pltpu.VMEM((CHUNK,), jnp.int32),
            pltpu.VMEM((CHUNK, D), jnp.float32),
            pltpu.SemaphoreType.DMA,
        ],
        compiler_params=pltpu.CompilerParams(use_tc_tiling_on_sc=False),
    )
    def k(idx_hbm, table_hbm, out_hbm, idx_v, rows_v, sem):
        wid = lax.axis_index("s") * NC + lax.axis_index("c")
        base = wid * n_per_w

        @pl.loop(0, n_chunks)
        def _chunk(g):
            off = base + g * CHUNK
            pltpu.sync_copy(idx_hbm.at[pl.ds(off, CHUNK)], idx_v)
            pltpu.async_copy(table_hbm.at[idx_v], rows_v, sem).wait()

            @pl.loop(0, CHUNK)
            def _scale(i):
                for j in range(D // L):
                    sl = pl.ds(j * L, L)
                    rows_v[i, sl] = rows_v[i, sl] * SCALE

            pltpu.sync_copy(rows_v, out_hbm.at[pl.ds(off, CHUNK)])

    return k


def kernel(x, table):
    B, S = x.shape
    V, D = table.shape
    idx = x.reshape(-1).astype(jnp.int32)
    out = _make_gather(V, D, B * S)(idx, table)
    return out.reshape(B, S, D)


# speedup vs baseline: 1.1329x; 1.1329x over previous
import functools
import math

import jax
import jax.numpy as jnp
from jax import lax
from jax.experimental import pallas as pl
from jax.experimental.pallas import tpu as pltpu
from jax.experimental.pallas import tpu_sc as plsc

D_MODEL = 64
SCALE = math.sqrt(D_MODEL)
CHUNK = 320
NBUF = 4


@functools.lru_cache(maxsize=None)
def _make_gather(V, D, B_total):
    info = plsc.get_sparse_core_info()
    NC, NS, L = info.num_cores, info.num_subcores, info.num_lanes
    NW = NC * NS
    assert B_total % NW == 0
    n_per_w = B_total // NW
    assert n_per_w % (CHUNK * NBUF) == 0
    n_chunks = n_per_w // CHUNK
    mesh = plsc.VectorSubcoreMesh(core_axis_name="c", subcore_axis_name="s")

    @functools.partial(
        pl.kernel,
        out_type=jax.ShapeDtypeStruct((B_total, D), jnp.float32),
        mesh=mesh,
        scratch_types=(
            [pltpu.VMEM((n_per_w,), jnp.int32)]
            + [pltpu.VMEM((CHUNK, D), jnp.float32) for _ in range(NBUF)]
            + [pltpu.SemaphoreType.DMA for _ in range(2 * NBUF)]
        ),
        compiler_params=pltpu.CompilerParams(use_tc_tiling_on_sc=False),
    )
    def k(idx_hbm, table_hbm, out_hbm, idx_all, *scratch):
        rows = scratch[:NBUF]
        gsem = scratch[NBUF : 2 * NBUF]
        ssem = scratch[2 * NBUF : 3 * NBUF]
        wid = lax.axis_index("s") * NC + lax.axis_index("c")
        base = wid * n_per_w

        pltpu.sync_copy(idx_hbm.at[pl.ds(base, n_per_w)], idx_all)

        def gather_start(c, b):
            pltpu.async_copy(
                table_hbm.at[idx_all.at[pl.ds(c * CHUNK, CHUNK)]], rows[b], gsem[b]
            )

        def gather_wait(c, b):
            pltpu.make_async_copy(
                table_hbm.at[idx_all.at[pl.ds(c * CHUNK, CHUNK)]], rows[b], gsem[b]
            ).wait()

        def store_start(c, b):
            pltpu.async_copy(rows[b], out_hbm.at[pl.ds(base + c * CHUNK, CHUNK)], ssem[b])

        def store_wait(c, b):
            pltpu.make_async_copy(
                rows[b], out_hbm.at[pl.ds(base + c * CHUNK, CHUNK)], ssem[b]
            ).wait()

        for b in range(NBUF):
            gather_start(b, b)

        @pl.loop(0, n_chunks, step=NBUF)
        def _step(g0):
            for b in range(NBUF):
                c = g0 + b
                gather_wait(c, b)

                @pl.loop(0, CHUNK, unroll=8)
                def _scale(i):
                    for j in range(D // L):
                        sl = pl.ds(j * L, L)
                        rows[b][i, sl] = rows[b][i, sl] * SCALE

                store_start(c, b)
            for b in range(NBUF):
                c = g0 + b
                store_wait(c, b)
                n = c + NBUF

                @pl.when(n < n_chunks)
                def _():
                    gather_start(n, b)

    return k


def kernel(x, table):
    B, S = x.shape
    V, D = table.shape
    idx = x.reshape(-1).astype(jnp.int32)
    out = _make_gather(V, D, B * S)(idx, table)
    return out.reshape(B, S, D)
